# Initial kernel scaffold; baseline (speedup 1.0000x reference)
import functools

import jax
import jax.numpy as jnp
from jax import lax
from jax.experimental import pallas as pl
from jax.experimental.pallas import tpu as pltpu
from jax.experimental.pallas import tpu_sc as plsc

N = 10000
NP = 10016
E = 320000
D = 64
K = 128
NW = 32
CH_W = 80
EP = NW * CH_W * K
RPT = NP // 16


def _tc_pre(x, wl_t, wr_t, b):
    def body(x_ref, wl_ref, wr_ref, b_ref, y_ref, r_ref):
        xv = x_ref[...]
        y_ref[...] = jnp.dot(xv, wl_ref[...], preferred_element_type=jnp.float32)
        r_ref[...] = jnp.dot(xv, wr_ref[...], preferred_element_type=jnp.float32) + b_ref[...]

    return pl.pallas_call(
        body,
        out_shape=[jax.ShapeDtypeStruct((NP, D), jnp.float32),
                   jax.ShapeDtypeStruct((NP, D), jnp.float32)],
    )(x, wl_t, wr_t, b)


def _tc_mid(agg, cnt, r, wl_t, wr_t, b):
    def body(agg_ref, cnt_ref, r_ref, wl_ref, wr_ref, b_ref, y_ref, r2_ref):
        a = agg_ref[0] + agg_ref[1]
        c = cnt_ref[0, :, 0:1] + cnt_ref[1, :, 0:1]
        h = jnp.maximum(a / jnp.maximum(c, 1.0) + r_ref[...], 0.0)
        y_ref[...] = jnp.dot(h, wl_ref[...], preferred_element_type=jnp.float32)
        r2_ref[...] = jnp.dot(h, wr_ref[...], preferred_element_type=jnp.float32) + b_ref[...]

    return pl.pallas_call(
        body,
        out_shape=[jax.ShapeDtypeStruct((NP, D), jnp.float32),
                   jax.ShapeDtypeStruct((NP, D), jnp.float32)],
    )(agg, cnt, r, wl_t, wr_t, b)


def _tc_fin(agg, cnt, r, w_t, b):
    def body(agg_ref, cnt_ref, r_ref, w_ref, b_ref, o_ref):
        a = agg_ref[0] + agg_ref[1]
        c = cnt_ref[0, :, 0:1] + cnt_ref[1, :, 0:1]
        h = jnp.maximum(a / jnp.maximum(c, 1.0) + r_ref[...], 0.0)
        o_ref[...] = jnp.dot(h, w_ref[...], preferred_element_type=jnp.float32) + b_ref[...]

    return pl.pallas_call(
        body,
        out_shape=jax.ShapeDtypeStruct((NP, 1), jnp.float32),
    )(agg, cnt, r, w_t, b)


def _make_sc_agg(with_cnt):
    mesh = plsc.VectorSubcoreMesh(core_axis_name="c", subcore_axis_name="s")
    out_types = [jax.ShapeDtypeStruct((2, NP, D), jnp.float32)]
    scratch = [
        pltpu.VMEM_SHARED((NP, D), jnp.float32),
        pltpu.VMEM((CH_W, K), jnp.int32),
        pltpu.VMEM((CH_W, K), jnp.int32),
        pltpu.VMEM((K, D), jnp.float32),
        pltpu.SemaphoreType.DMA,
    ]
    if with_cnt:
        out_types.append(jax.ShapeDtypeStruct((2, NP, 16), jnp.float32))
        scratch += [
            pltpu.VMEM_SHARED((NP, 16), jnp.float32),
            pltpu.VMEM((K, 16), jnp.float32),
        ]

    @functools.partial(pl.kernel, out_type=out_types, mesh=mesh,
                       scratch_types=scratch)
    def k(*refs):
        if with_cnt:
            (y_hbm, src_hbm, dst_hbm, z64_hbm, z16_hbm, ones_hbm,
             agg_out, cnt_out,
             agg_sh, src_v, dst_v, rows_v, sem, cnt_sh, ones_v) = refs
        else:
            (y_hbm, src_hbm, dst_hbm, z64_hbm,
             agg_out,
             agg_sh, src_v, dst_v, rows_v, sem) = refs
        c = lax.axis_index("c")
        s = lax.axis_index("s")
        wid = c * 16 + s

        pltpu.sync_copy(z64_hbm.at[pl.ds(s * RPT, RPT)],
                        agg_sh.at[pl.ds(s * RPT, RPT)])
        pltpu.sync_copy(src_hbm.at[pl.ds(wid * CH_W, CH_W)], src_v)
        pltpu.sync_copy(dst_hbm.at[pl.ds(wid * CH_W, CH_W)], dst_v)
        if with_cnt:
            pltpu.sync_copy(z16_hbm.at[pl.ds(s * RPT, RPT)],
                            cnt_sh.at[pl.ds(s * RPT, RPT)])
            pltpu.sync_copy(ones_hbm, ones_v)
        plsc.subcore_barrier()

        @pl.loop(0, CH_W)
        def _(t):
            pltpu.async_copy(y_hbm.at[src_v.at[t]], rows_v, sem).wait()
            pltpu.sync_copy(rows_v, agg_sh.at[dst_v.at[t]], add=True)
            if with_cnt:
                pltpu.sync_copy(ones_v, cnt_sh.at[dst_v.at[t]], add=True)

        plsc.subcore_barrier()
        pltpu.sync_copy(agg_sh.at[pl.ds(s * RPT, RPT)],
                        agg_out.at[c, pl.ds(s * RPT, RPT)])
        if with_cnt:
            pltpu.sync_copy(cnt_sh.at[pl.ds(s * RPT, RPT)],
                            cnt_out.at[c, pl.ds(s * RPT, RPT)])

    return k


_sc_agg_cnt = _make_sc_agg(with_cnt=True)
_sc_agg = _make_sc_agg(with_cnt=False)


def kernel(x, edge_index, W1l, b1, W1r, W2l, b2, W2r, Wlin, blin):
    src = edge_index[0].astype(jnp.int32)
    dst = edge_index[1].astype(jnp.int32)
    pad = jnp.full((EP - E,), N, jnp.int32)
    src2d = jnp.concatenate([src, pad]).reshape(EP // K, K)
    dst2d = jnp.concatenate([dst, pad]).reshape(EP // K, K)
    xp = jnp.pad(x, ((0, NP - N), (0, 0)))
    z64 = jnp.zeros((NP, D), jnp.float32)
    z16 = jnp.zeros((NP, 16), jnp.float32)
    ones16 = jnp.ones((K, 16), jnp.float32)

    y1, r1 = _tc_pre(xp, W1l.T, W1r.T, b1.reshape(1, D))
    agg1, cnt = _sc_agg_cnt(y1, src2d, dst2d, z64, z16, ones16)
    y2, r2 = _tc_mid(agg1, cnt, r1, W2l.T, W2r.T, b2.reshape(1, D))
    agg2 = _sc_agg(y2, src2d, dst2d, z64)
    out = _tc_fin(agg2, cnt, r2, Wlin.T, blin.reshape(1, 1))
    return out[:N]


# speedup vs baseline: 3.2507x; 3.2507x over previous
import functools

import jax
import jax.numpy as jnp
from jax import lax
from jax.experimental import pallas as pl
from jax.experimental.pallas import tpu as pltpu
from jax.experimental.pallas import tpu_sc as plsc

N = 10000
NP = 10112
E = 320000
D = 64
W = 128
K = 128
NW = 32
CH_W = 80
EP = NW * CH_W * K
RPT = NP // 16


def _tc_pre(x, wl_t, wr_t, b):
    def body(x_ref, wl_ref, wr_ref, b_ref, y_ref, r_ref):
        xv = x_ref[...]
        y_ref[:, 0:D] = jnp.dot(xv, wl_ref[...], preferred_element_type=jnp.float32)
        y_ref[:, D:W] = jnp.ones((NP, W - D), jnp.float32)
        r_ref[...] = jnp.dot(xv, wr_ref[...], preferred_element_type=jnp.float32) + b_ref[...]

    return pl.pallas_call(
        body,
        out_shape=[jax.ShapeDtypeStruct((NP, W), jnp.float32),
                   jax.ShapeDtypeStruct((NP, D), jnp.float32)],
    )(x, wl_t, wr_t, b)


def _tc_mid(agg, r, wl_t, wr_t, b):
    def body(agg_ref, r_ref, wl_ref, wr_ref, b_ref, y_ref, r2_ref):
        a = agg_ref[0] + agg_ref[1]
        c = a[:, D:D + 1]
        h = jnp.maximum(a[:, 0:D] / jnp.maximum(c, 1.0) + r_ref[...], 0.0)
        y_ref[:, 0:D] = jnp.dot(h, wl_ref[...], preferred_element_type=jnp.float32)
        y_ref[:, D:W] = jnp.ones((NP, W - D), jnp.float32)
        r2_ref[...] = jnp.dot(h, wr_ref[...], preferred_element_type=jnp.float32) + b_ref[...]

    return pl.pallas_call(
        body,
        out_shape=[jax.ShapeDtypeStruct((NP, W), jnp.float32),
                   jax.ShapeDtypeStruct((NP, D), jnp.float32)],
    )(agg, r, wl_t, wr_t, b)


def _tc_fin(agg, r, w_t, b):
    def body(agg_ref, r_ref, w_ref, b_ref, o_ref):
        a = agg_ref[0] + agg_ref[1]
        c = a[:, D:D + 1]
        h = jnp.maximum(a[:, 0:D] / jnp.maximum(c, 1.0) + r_ref[...], 0.0)
        o_ref[...] = jnp.dot(h, w_ref[...], preferred_element_type=jnp.float32) + b_ref[...]

    return pl.pallas_call(
        body,
        out_shape=jax.ShapeDtypeStruct((NP, 1), jnp.float32),
    )(agg, r, w_t, b)


@functools.lru_cache(maxsize=None)
def _make_sc_agg():
    mesh = plsc.VectorSubcoreMesh(core_axis_name="c", subcore_axis_name="s")
    scratch = [
        pltpu.VMEM_SHARED((NP, W), jnp.float32),
        pltpu.VMEM((CH_W, K), jnp.int32),
        pltpu.VMEM((CH_W, K), jnp.int32),
        pltpu.VMEM((K, W), jnp.float32),
        pltpu.SemaphoreType.DMA,
    ]

    @functools.partial(
        pl.kernel, mesh=mesh, scratch_types=scratch,
        out_type=jax.ShapeDtypeStruct((2, NP, W), jnp.float32))
    def k(y_hbm, src_hbm, dst_hbm, z_hbm, agg_out,
          acc_sh, src_v, dst_v, rows_v, sem):
        c = lax.axis_index("c")
        s = lax.axis_index("s")
        wid = c * 16 + s

        pltpu.sync_copy(z_hbm.at[pl.ds(s * RPT, RPT)],
                        acc_sh.at[pl.ds(s * RPT, RPT)])
        pltpu.sync_copy(src_hbm.at[pl.ds(wid * CH_W, CH_W)], src_v)
        pltpu.sync_copy(dst_hbm.at[pl.ds(wid * CH_W, CH_W)], dst_v)
        plsc.subcore_barrier()

        @pl.loop(0, CH_W)
        def _(t):
            pltpu.async_copy(y_hbm.at[src_v.at[t]], rows_v, sem).wait()
            pltpu.sync_copy(rows_v, acc_sh.at[dst_v.at[t]], add=True)

        plsc.subcore_barrier()
        pltpu.sync_copy(acc_sh.at[pl.ds(s * RPT, RPT)],
                        agg_out.at[c, pl.ds(s * RPT, RPT)])

    return k


def kernel(x, edge_index, W1l, b1, W1r, W2l, b2, W2r, Wlin, blin):
    src = edge_index[0].astype(jnp.int32)
    dst = edge_index[1].astype(jnp.int32)
    pad = jnp.full((EP - E,), N, jnp.int32)
    src2d = jnp.concatenate([src, pad]).reshape(EP // K, K)
    dst2d = jnp.concatenate([dst, pad]).reshape(EP // K, K)
    xp = jnp.pad(x, ((0, NP - N), (0, 0)))
    z = jnp.zeros((NP, W), jnp.float32)

    sc_agg = _make_sc_agg()
    y1, r1 = _tc_pre(xp, W1l.T, W1r.T, b1.reshape(1, D))
    agg1 = sc_agg(y1, src2d, dst2d, z)
    y2, r2 = _tc_mid(agg1, r1, W2l.T, W2r.T, b2.reshape(1, D))
    agg2 = sc_agg(y2, src2d, dst2d, z)
    out = _tc_fin(agg2, r2, Wlin.T, blin.reshape(1, 1))
    return out[:N]


# speedup vs baseline: 3.5963x; 1.1063x over previous
import functools

import jax
import jax.numpy as jnp
from jax import lax
from jax.experimental import pallas as pl
from jax.experimental.pallas import tpu as pltpu
from jax.experimental.pallas import tpu_sc as plsc

N = 10000
NP = 10112
E = 320000
D = 64
W = 128
K = 128
KE = 128
NW = 32
CH_W = 80
EP = NW * CH_W * K
RPT = NP // 16


def _tc_pre(x, wl_t, wr_t, b):
    def body(x_ref, wl_ref, wr_ref, b_ref, y_ref, r_ref):
        xv = x_ref[...]
        y_ref[:, 0:D] = jnp.dot(xv, wl_ref[...], preferred_element_type=jnp.float32)
        y_ref[:, D:W] = jnp.ones((NP, W - D), jnp.float32)
        r_ref[...] = jnp.dot(xv, wr_ref[...], preferred_element_type=jnp.float32) + b_ref[...]

    return pl.pallas_call(
        body,
        out_shape=[jax.ShapeDtypeStruct((NP, W), jnp.float32),
                   jax.ShapeDtypeStruct((NP, D), jnp.float32)],
    )(x, wl_t, wr_t, b)


def _tc_mid(agg, r, wl_t, wr_t, b):
    def body(agg_ref, r_ref, wl_ref, wr_ref, b_ref, y_ref, r2_ref):
        a = agg_ref[0] + agg_ref[1]
        c = a[:, D:D + 1]
        h = jnp.maximum(a[:, 0:D] / jnp.maximum(c, 1.0) + r_ref[...], 0.0)
        y_ref[:, 0:D] = jnp.dot(h, wl_ref[...], preferred_element_type=jnp.float32)
        y_ref[:, D:W] = jnp.ones((NP, W - D), jnp.float32)
        r2_ref[...] = jnp.dot(h, wr_ref[...], preferred_element_type=jnp.float32) + b_ref[...]

    return pl.pallas_call(
        body,
        out_shape=[jax.ShapeDtypeStruct((NP, W), jnp.float32),
                   jax.ShapeDtypeStruct((NP, D), jnp.float32)],
    )(agg, r, wl_t, wr_t, b)


def _tc_fin(agg, r, w_t, b):
    def body(agg_ref, r_ref, w_ref, b_ref, o_ref):
        a = agg_ref[0] + agg_ref[1]
        c = a[:, D:D + 1]
        h = jnp.maximum(a[:, 0:D] / jnp.maximum(c, 1.0) + r_ref[...], 0.0)
        o_ref[...] = jnp.dot(h, w_ref[...], preferred_element_type=jnp.float32) + b_ref[...]

    return pl.pallas_call(
        body,
        out_shape=jax.ShapeDtypeStruct((NP, 1), jnp.float32),
    )(agg, r, w_t, b)


@functools.lru_cache(maxsize=None)
def _make_sc_agg():
    mesh = plsc.VectorSubcoreMesh(core_axis_name="c", subcore_axis_name="s")
    NCH = CH_W * K // KE
    PH = 2
    CPP = NCH // PH
    scratch = [
        pltpu.VMEM_SHARED((NP, W), jnp.float32),
        pltpu.VMEM((CPP, KE), jnp.int32),
        pltpu.VMEM((CPP, KE), jnp.int32),
        pltpu.VMEM((2, KE, W), jnp.float32),
        pltpu.SemaphoreType.DMA((2,)),
        pltpu.SemaphoreType.DMA((2,)),
    ]

    @functools.partial(
        pl.kernel, mesh=mesh, scratch_types=scratch,
        out_type=jax.ShapeDtypeStruct((2, NP, W), jnp.float32))
    def k(y_hbm, src_hbm, dst_hbm, z_hbm, agg_out,
          acc_sh, src_v, dst_v, rows_v, semg, sems):
        c = lax.axis_index("c")
        s = lax.axis_index("s")
        wid = c * 16 + s

        def gather(t, b):
            return pltpu.make_async_copy(
                y_hbm.at[src_v.at[t]], rows_v.at[b], semg.at[b])

        def scatter(t, b):
            return pltpu.make_async_copy(
                rows_v.at[b], acc_sh.at[dst_v.at[t]], sems.at[b])

        pltpu.sync_copy(z_hbm.at[pl.ds(s * RPT, RPT)],
                        acc_sh.at[pl.ds(s * RPT, RPT)])
        plsc.subcore_barrier()

        for phase in range(PH):
            base = wid * NCH + phase * CPP
            pltpu.sync_copy(src_hbm.at[pl.ds(base, CPP)], src_v)
            pltpu.sync_copy(dst_hbm.at[pl.ds(base, CPP)], dst_v)

            for b in range(2):
                gather(b, b).start()

            @pl.loop(0, CPP, step=2)
            def _(t0):
                for b in range(2):
                    t = t0 + b
                    gather(t, b).wait()
                    scatter(t, b).start(add=True)
                    scatter(t, b).wait()

                    @pl.when(t + 2 < CPP)
                    def _():
                        gather(t + 2, b).start()

        plsc.subcore_barrier()
        pltpu.sync_copy(acc_sh.at[pl.ds(s * RPT, RPT)],
                        agg_out.at[c, pl.ds(s * RPT, RPT)])

    return k


def kernel(x, edge_index, W1l, b1, W1r, W2l, b2, W2r, Wlin, blin):
    src = edge_index[0].astype(jnp.int32)
    dst = edge_index[1].astype(jnp.int32)
    pad = jnp.full((EP - E,), N, jnp.int32)
    src2d = jnp.concatenate([src, pad]).reshape(EP // KE, KE)
    dst2d = jnp.concatenate([dst, pad]).reshape(EP // KE, KE)
    xp = jnp.pad(x, ((0, NP - N), (0, 0)))
    z = jnp.zeros((NP, W), jnp.float32)

    sc_agg = _make_sc_agg()
    y1, r1 = _tc_pre(xp, W1l.T, W1r.T, b1.reshape(1, D))
    agg1 = sc_agg(y1, src2d, dst2d, z)
    y2, r2 = _tc_mid(agg1, r1, W2l.T, W2r.T, b2.reshape(1, D))
    agg2 = sc_agg(y2, src2d, dst2d, z)
    out = _tc_fin(agg2, r2, Wlin.T, blin.reshape(1, 1))
    return out[:N]


# speedup vs baseline: 12.6058x; 3.5052x over previous
import functools

import jax
import jax.numpy as jnp
from jax import lax
from jax.experimental import pallas as pl
from jax.experimental.pallas import tpu as pltpu
from jax.experimental.pallas import tpu_sc as plsc

N = 10000
NP = 10112
E = 320000
D = 64
W = 128
K = 128
KE = 128
NW = 32
CH_W = 80
EP = NW * CH_W * K
RPT = NP // 16


def _tc_pre(x, wl_t, wr_t, b):
    def body(x_ref, wl_ref, wr_ref, b_ref, y_ref, r_ref):
        xv = x_ref[...]
        y_ref[:, 0:D] = jnp.dot(xv, wl_ref[...], preferred_element_type=jnp.float32)
        y_ref[:, D:W] = jnp.ones((NP, W - D), jnp.float32)
        r_ref[...] = jnp.dot(xv, wr_ref[...], preferred_element_type=jnp.float32) + b_ref[...]

    return pl.pallas_call(
        body,
        out_shape=[jax.ShapeDtypeStruct((NP, W), jnp.float32),
                   jax.ShapeDtypeStruct((NP, D), jnp.float32)],
    )(x, wl_t, wr_t, b)


def _tc_mid(agg, r, wl_t, wr_t, b):
    def body(agg_ref, r_ref, wl_ref, wr_ref, b_ref, y_ref, r2_ref):
        a = agg_ref[0] + agg_ref[1]
        c = a[:, D:D + 1]
        h = jnp.maximum(a[:, 0:D] / jnp.maximum(c, 1.0) + r_ref[...], 0.0)
        y_ref[:, 0:D] = jnp.dot(h, wl_ref[...], preferred_element_type=jnp.float32)
        y_ref[:, D:W] = jnp.ones((NP, W - D), jnp.float32)
        r2_ref[...] = jnp.dot(h, wr_ref[...], preferred_element_type=jnp.float32) + b_ref[...]

    return pl.pallas_call(
        body,
        out_shape=[jax.ShapeDtypeStruct((NP, W), jnp.float32),
                   jax.ShapeDtypeStruct((NP, D), jnp.float32)],
    )(agg, r, wl_t, wr_t, b)


def _tc_fin(agg, r, w_t, b):
    def body(agg_ref, r_ref, w_ref, b_ref, o_ref):
        a = agg_ref[0] + agg_ref[1]
        c = a[:, D:D + 1]
        h = jnp.maximum(a[:, 0:D] / jnp.maximum(c, 1.0) + r_ref[...], 0.0)
        o_ref[...] = jnp.dot(h, w_ref[...], preferred_element_type=jnp.float32) + b_ref[...]

    return pl.pallas_call(
        body,
        out_shape=jax.ShapeDtypeStruct((NP, 1), jnp.float32),
    )(agg, r, w_t, b)


@functools.lru_cache(maxsize=None)
def _make_sc_agg():
    mesh = plsc.VectorSubcoreMesh(core_axis_name="c", subcore_axis_name="s")
    NCH = CH_W * K // KE
    PH = 2
    CPP = NCH // PH
    scratch = [
        pltpu.VMEM_SHARED((NP, W), jnp.float32),
        pltpu.VMEM((CPP, KE), jnp.int32),
        pltpu.VMEM((CPP, KE), jnp.int32),
        pltpu.VMEM((2, KE, W), jnp.float32),
        pltpu.SemaphoreType.DMA((2,)),
        pltpu.SemaphoreType.DMA((2,)),
    ]

    @functools.partial(
        pl.kernel, mesh=mesh, scratch_types=scratch,
        out_type=jax.ShapeDtypeStruct((2, NP, W), jnp.float32))
    def k(y_hbm, src_hbm, dst_hbm, z_hbm, agg_out,
          acc_sh, src_v, dst_v, rows_v, semg, sems):
        c = lax.axis_index("c")
        s = lax.axis_index("s")
        wid = c * 16 + s

        def gather(t, b):
            return pltpu.make_async_copy(
                y_hbm.at[src_v.at[t]], rows_v.at[b], semg.at[b])

        def scatter(t, b):
            return pltpu.make_async_copy(
                rows_v.at[b], acc_sh.at[dst_v.at[t]], sems.at[b])

        pltpu.sync_copy(z_hbm.at[pl.ds(s * RPT, RPT)],
                        acc_sh.at[pl.ds(s * RPT, RPT)])
        plsc.subcore_barrier()

        for phase in range(PH):
            base = wid * NCH + phase * CPP
            pltpu.sync_copy(src_hbm.at[pl.ds(base, CPP)], src_v)
            pltpu.sync_copy(dst_hbm.at[pl.ds(base, CPP)], dst_v)

            for b in range(2):
                gather(b, b).start()

            @pl.loop(0, CPP, step=2)
            def _(t0):
                for b in range(2):
                    t = t0 + b
                    gather(t, b).wait()
                    scatter(t, b).start(add=True)
                    scatter(t, b).wait()

                    @pl.when(t + 2 < CPP)
                    def _():
                        gather(t + 2, b).start()

        plsc.subcore_barrier()
        pltpu.sync_copy(acc_sh.at[pl.ds(s * RPT, RPT)],
                        agg_out.at[c, pl.ds(s * RPT, RPT)])

    return k


def kernel(x, edge_index, W1l, b1, W1r, W2l, b2, W2r, Wlin, blin):
    src = edge_index[0].astype(jnp.int32)
    dst = edge_index[1].astype(jnp.int32)
    pad = N + jnp.arange(EP - E, dtype=jnp.int32) % (NP - N)
    src2d = jnp.concatenate([src, pad]).reshape(EP // KE, KE)
    dst2d = jnp.concatenate([dst, pad]).reshape(EP // KE, KE)
    xp = jnp.pad(x, ((0, NP - N), (0, 0)))
    z = jnp.zeros((NP, W), jnp.float32)

    sc_agg = _make_sc_agg()
    y1, r1 = _tc_pre(xp, W1l.T, W1r.T, b1.reshape(1, D))
    agg1 = sc_agg(y1, src2d, dst2d, z)
    y2, r2 = _tc_mid(agg1, r1, W2l.T, W2r.T, b2.reshape(1, D))
    agg2 = sc_agg(y2, src2d, dst2d, z)
    out = _tc_fin(agg2, r2, Wlin.T, blin.reshape(1, 1))
    return out[:N]


# speedup vs baseline: 12.9520x; 1.0275x over previous
import functools

import jax
import jax.numpy as jnp
from jax import lax
from jax.experimental import pallas as pl
from jax.experimental.pallas import tpu as pltpu
from jax.experimental.pallas import tpu_sc as plsc

N = 10000
NP = 10112
E = 320000
D = 64
W = 128
WA = 80
K = 128
KE = 128
NW = 32
CH_W = 80
EP = NW * CH_W * K
RPT = NP // 16


def _tc_pre(x, wl_t, wr_t, b):
    def body(x_ref, wl_ref, wr_ref, b_ref, y_ref, r_ref):
        xv = x_ref[...]
        y_ref[:, 0:D] = jnp.dot(xv, wl_ref[...], preferred_element_type=jnp.float32)
        y_ref[:, D:W] = jnp.ones((NP, W - D), jnp.float32)
        r_ref[...] = jnp.dot(xv, wr_ref[...], preferred_element_type=jnp.float32) + b_ref[...]

    return pl.pallas_call(
        body,
        out_shape=[jax.ShapeDtypeStruct((NP, W), jnp.float32),
                   jax.ShapeDtypeStruct((NP, D), jnp.float32)],
    )(x, wl_t, wr_t, b)


def _tc_mid(agg, r, wl_t, wr_t, b):
    def body(agg_ref, r_ref, wl_ref, wr_ref, b_ref, y_ref, r2_ref):
        a = agg_ref[0] + agg_ref[1]
        c = a[:, D:D + 1]
        h = jnp.maximum(a[:, 0:D] / jnp.maximum(c, 1.0) + r_ref[...], 0.0)
        y_ref[:, 0:D] = jnp.dot(h, wl_ref[...], preferred_element_type=jnp.float32)
        y_ref[:, D:W] = jnp.ones((NP, W - D), jnp.float32)
        r2_ref[...] = jnp.dot(h, wr_ref[...], preferred_element_type=jnp.float32) + b_ref[...]

    return pl.pallas_call(
        body,
        out_shape=[jax.ShapeDtypeStruct((NP, W), jnp.float32),
                   jax.ShapeDtypeStruct((NP, D), jnp.float32)],
    )(agg, r, wl_t, wr_t, b)


def _tc_fin(agg, r, w_t, b):
    def body(agg_ref, r_ref, w_ref, b_ref, o_ref):
        a = agg_ref[0] + agg_ref[1]
        c = a[:, D:D + 1]
        h = jnp.maximum(a[:, 0:D] / jnp.maximum(c, 1.0) + r_ref[...], 0.0)
        o_ref[...] = jnp.dot(h, w_ref[...], preferred_element_type=jnp.float32) + b_ref[...]

    return pl.pallas_call(
        body,
        out_shape=jax.ShapeDtypeStruct((NP, 1), jnp.float32),
    )(agg, r, w_t, b)


@functools.lru_cache(maxsize=None)
def _make_sc_agg():
    mesh = plsc.VectorSubcoreMesh(core_axis_name="c", subcore_axis_name="s")
    NCH = CH_W * K // KE
    PH = 2
    CPP = NCH // PH
    scratch = [
        pltpu.VMEM_SHARED((NP, W), jnp.float32),
        pltpu.VMEM((CPP, KE), jnp.int32),
        pltpu.VMEM((CPP, KE), jnp.int32),
        pltpu.VMEM((2, KE, W), jnp.float32),
        pltpu.SemaphoreType.DMA((2,)),
        pltpu.SemaphoreType.DMA((2,)),
    ]

    @functools.partial(
        pl.kernel, mesh=mesh, scratch_types=scratch,
        out_type=jax.ShapeDtypeStruct((2, NP, W), jnp.float32))
    def k(y_hbm, src_hbm, dst_hbm, agg_out,
          acc_sh, src_v, dst_v, rows_v, semg, sems):
        c = lax.axis_index("c")
        s = lax.axis_index("s")
        wid = c * 16 + s

        def gather(t, b):
            return pltpu.make_async_copy(
                y_hbm.at[src_v.at[t]], rows_v.at[b], semg.at[b])

        def scatter(t, b):
            return pltpu.make_async_copy(
                rows_v.at[b], acc_sh.at[dst_v.at[t]], sems.at[b])

        @pl.loop(0, KE)
        def _(i):
            for j in range(W // 16):
                rows_v[0, i, pl.ds(j * 16, 16)] = jnp.zeros((16,), jnp.float32)

        for j in range(4):
            pltpu.sync_copy(rows_v.at[0],
                            acc_sh.at[pl.ds(s * RPT + j * KE, KE)])
        pltpu.sync_copy(rows_v.at[0, pl.ds(0, RPT - 4 * KE)],
                        acc_sh.at[pl.ds(s * RPT + 4 * KE, RPT - 4 * KE)])
        plsc.subcore_barrier()

        for phase in range(PH):
            base = wid * NCH + phase * CPP
            pltpu.sync_copy(src_hbm.at[pl.ds(base, CPP)], src_v)
            pltpu.sync_copy(dst_hbm.at[pl.ds(base, CPP)], dst_v)

            for b in range(2):
                gather(b, b).start()

            @pl.loop(0, CPP, step=2)
            def _(t0):
                for b in range(2):
                    t = t0 + b
                    gather(t, b).wait()
                    scatter(t, b).start(add=True)
                    scatter(t, b).wait()

                    @pl.when(t + 2 < CPP)
                    def _():
                        gather(t + 2, b).start()

        plsc.subcore_barrier()
        pltpu.sync_copy(acc_sh.at[pl.ds(s * RPT, RPT)],
                        agg_out.at[c, pl.ds(s * RPT, RPT)])

    return k


def kernel(x, edge_index, W1l, b1, W1r, W2l, b2, W2r, Wlin, blin):
    src = edge_index[0].astype(jnp.int32)
    dst = edge_index[1].astype(jnp.int32)
    pad = N + jnp.arange(EP - E, dtype=jnp.int32) % (NP - N)
    src2d = jnp.concatenate([src, pad]).reshape(EP // KE, KE)
    dst2d = jnp.concatenate([dst, pad]).reshape(EP // KE, KE)
    xp = jnp.pad(x, ((0, NP - N), (0, 0)))

    sc_agg = _make_sc_agg()
    y1, r1 = _tc_pre(xp, W1l.T, W1r.T, b1.reshape(1, D))
    agg1 = sc_agg(y1, src2d, dst2d)
    y2, r2 = _tc_mid(agg1, r1, W2l.T, W2r.T, b2.reshape(1, D))
    agg2 = sc_agg(y2, src2d, dst2d)
    out = _tc_fin(agg2, r2, Wlin.T, blin.reshape(1, 1))
    return out[:N]


# speedup vs baseline: 13.1723x; 1.0170x over previous
import functools

import jax
import jax.numpy as jnp
from jax import lax
from jax.experimental import pallas as pl
from jax.experimental.pallas import tpu as pltpu
from jax.experimental.pallas import tpu_sc as plsc

N = 10000
NP = 10112
E = 320000
D = 64
W = 128
WA = 80
K = 128
KE = 128
NW = 32
CH_W = 80
EP = NW * CH_W * K
RPT = NP // 16


def _tc_pre(x, wl_t, wr_t, b):
    def body(x_ref, wl_ref, wr_ref, b_ref, y_ref, r_ref):
        xv = x_ref[...]
        y_ref[0:N, 0:D] = jnp.dot(xv, wl_ref[...], preferred_element_type=jnp.float32)
        y_ref[:, D:W] = jnp.ones((NP, W - D), jnp.float32)
        r_ref[0:N, :] = jnp.dot(xv, wr_ref[...], preferred_element_type=jnp.float32) + b_ref[...]
        r_ref[N:NP, :] = jnp.zeros((NP - N, D), jnp.float32)

    return pl.pallas_call(
        body,
        out_shape=[jax.ShapeDtypeStruct((NP, W), jnp.float32),
                   jax.ShapeDtypeStruct((NP, D), jnp.float32)],
    )(x, wl_t, wr_t, b)


def _tc_mid(agg, r, wl_t, wr_t, b):
    def body(agg_ref, r_ref, wl_ref, wr_ref, b_ref, y_ref, r2_ref):
        a = agg_ref[0] + agg_ref[1]
        c = a[:, D:D + 1]
        h = jnp.maximum(a[:, 0:D] / jnp.maximum(c, 1.0) + r_ref[...], 0.0)
        y_ref[:, 0:D] = jnp.dot(h, wl_ref[...], preferred_element_type=jnp.float32)
        y_ref[:, D:W] = jnp.ones((NP, W - D), jnp.float32)
        r2_ref[...] = jnp.dot(h, wr_ref[...], preferred_element_type=jnp.float32) + b_ref[...]

    return pl.pallas_call(
        body,
        out_shape=[jax.ShapeDtypeStruct((NP, W), jnp.float32),
                   jax.ShapeDtypeStruct((NP, D), jnp.float32)],
    )(agg, r, wl_t, wr_t, b)


def _tc_fin(agg, r, w_t, b):
    def body(agg_ref, r_ref, w_ref, b_ref, o_ref):
        a = agg_ref[0] + agg_ref[1]
        c = a[:, D:D + 1]
        h = jnp.maximum(a[:, 0:D] / jnp.maximum(c, 1.0) + r_ref[...], 0.0)
        o_ref[...] = jnp.dot(h, w_ref[...], preferred_element_type=jnp.float32) + b_ref[...]

    return pl.pallas_call(
        body,
        out_shape=jax.ShapeDtypeStruct((NP, 1), jnp.float32),
    )(agg, r, w_t, b)


@functools.lru_cache(maxsize=None)
def _make_sc_agg():
    mesh = plsc.VectorSubcoreMesh(core_axis_name="c", subcore_axis_name="s")
    NCH = CH_W * K // KE
    PH = 2
    CPP = NCH // PH
    scratch = [
        pltpu.VMEM_SHARED((NP, W), jnp.float32),
        pltpu.VMEM((CPP, KE), jnp.int32),
        pltpu.VMEM((CPP, KE), jnp.int32),
        pltpu.VMEM((2, KE, W), jnp.float32),
        pltpu.SemaphoreType.DMA((2,)),
        pltpu.SemaphoreType.DMA((2,)),
    ]

    @functools.partial(
        pl.kernel, mesh=mesh, scratch_types=scratch,
        out_type=jax.ShapeDtypeStruct((2, NP, W), jnp.float32))
    def k(y_hbm, src_hbm, dst_hbm, agg_out,
          acc_sh, src_v, dst_v, rows_v, semg, sems):
        c = lax.axis_index("c")
        s = lax.axis_index("s")
        wid = c * 16 + s

        def gather(t, b):
            return pltpu.make_async_copy(
                y_hbm.at[src_v.at[t]], rows_v.at[b], semg.at[b])

        def scatter(t, b):
            return pltpu.make_async_copy(
                rows_v.at[b], acc_sh.at[dst_v.at[t]], sems.at[b])

        @pl.loop(0, KE)
        def _(i):
            for j in range(W // 16):
                rows_v[0, i, pl.ds(j * 16, 16)] = jnp.zeros((16,), jnp.float32)

        for j in range(4):
            pltpu.sync_copy(rows_v.at[0],
                            acc_sh.at[pl.ds(s * RPT + j * KE, KE)])
        pltpu.sync_copy(rows_v.at[0, pl.ds(0, RPT - 4 * KE)],
                        acc_sh.at[pl.ds(s * RPT + 4 * KE, RPT - 4 * KE)])
        plsc.subcore_barrier()

        for phase in range(PH):
            base = wid * NCH + phase * CPP
            pltpu.sync_copy(src_hbm.at[pl.ds(base, CPP)], src_v)
            pltpu.sync_copy(dst_hbm.at[pl.ds(base, CPP)], dst_v)

            for b in range(2):
                gather(b, b).start()

            @pl.loop(0, CPP, step=2)
            def _(t0):
                for b in range(2):
                    t = t0 + b
                    gather(t, b).wait()
                    scatter(t, b).start(add=True)
                    scatter(t, b).wait()

                    @pl.when(t + 2 < CPP)
                    def _():
                        gather(t + 2, b).start()

        plsc.subcore_barrier()
        pltpu.sync_copy(acc_sh.at[pl.ds(s * RPT, RPT)],
                        agg_out.at[c, pl.ds(s * RPT, RPT)])

    return k


def kernel(x, edge_index, W1l, b1, W1r, W2l, b2, W2r, Wlin, blin):
    src = edge_index[0].astype(jnp.int32)
    dst = edge_index[1].astype(jnp.int32)
    pad = N + jnp.arange(EP - E, dtype=jnp.int32) % (NP - N)
    src2d = jnp.concatenate([src, pad]).reshape(EP // KE, KE)
    dst2d = jnp.concatenate([dst, pad]).reshape(EP // KE, KE)
    sc_agg = _make_sc_agg()
    y1, r1 = _tc_pre(x, W1l.T, W1r.T, b1.reshape(1, D))
    agg1 = sc_agg(y1, src2d, dst2d)
    y2, r2 = _tc_mid(agg1, r1, W2l.T, W2r.T, b2.reshape(1, D))
    agg2 = sc_agg(y2, src2d, dst2d)
    out = _tc_fin(agg2, r2, Wlin.T, blin.reshape(1, 1))
    return out[:N]
